# Initial kernel scaffold; baseline (speedup 1.0000x reference)
import jax
import jax.numpy as jnp
from jax.experimental import pallas as pl


def kernel(x, edge_index, W1, a_src1, a_dst1, b1, W2, a_src2, a_dst2, b2):
    raise NotImplementedError("write your pallas kernel here")


# speedup vs baseline: 23.7583x; 23.7583x over previous
import dataclasses
import functools

import jax
import jax.numpy as jnp
from jax import lax
from jax.experimental import pallas as pl
from jax.experimental.pallas import tpu as pltpu
from jax.experimental.pallas import tpu_sc as plsc

N = 10000
D = 128
E = 320000
NC = 2
NS = 16
NW = NC * NS
EPW = E // NW
CHUNK = 80
NCHUNK = EPW // CHUNK
NPAD = 10240
STRIPE = NPAD // NS


def _proj_body(x_ref, w_ref, asv_ref, adv_ref, h_ref, as_ref, ad_ref, c_ref):
    h = jnp.dot(x_ref[...], w_ref[...], preferred_element_type=jnp.float32)
    h_ref[...] = h
    a_s = jnp.dot(h, asv_ref[...], preferred_element_type=jnp.float32)
    a_d = jnp.dot(h, adv_ref[...], preferred_element_type=jnp.float32)
    as_ref[...] = a_s
    ad_ref[...] = a_d
    c = jnp.maximum(jnp.max(a_s) + jnp.max(a_d), 0.0)
    c_ref[...] = jnp.reshape(c, (1, 1))


def _proj(x, w, asv, adv):
    return pl.pallas_call(
        _proj_body,
        out_shape=(
            jax.ShapeDtypeStruct((N, D), jnp.float32),
            jax.ShapeDtypeStruct((N, 1), jnp.float32),
            jax.ShapeDtypeStruct((N, 1), jnp.float32),
            jax.ShapeDtypeStruct((1, 1), jnp.float32),
        ),
    )(x, w, asv, adv)


def _combine_body(accp_ref, dp_ref, h_ref, asv_ref, adv_ref, c_ref, b_ref,
                  z_ref, *, relu):
    h = h_ref[...]
    es = (jnp.dot(h, asv_ref[...], preferred_element_type=jnp.float32)
          + jnp.dot(h, adv_ref[...], preferred_element_type=jnp.float32))
    es = jnp.where(es >= 0.0, es, 0.2 * es)
    es = jnp.exp(es - c_ref[...])
    acc = accp_ref[0, :N, :] + accp_ref[1, :N, :] + es * h
    den = dp_ref[0, :N, :] + dp_ref[1, :N, :] + es + 1e-16
    z = acc / den + b_ref[...]
    if relu:
        z = jnp.maximum(z, 0.0)
    z_ref[...] = z


def _combine(accp, dp3, h, asv, adv, c, b2d, relu):
    return pl.pallas_call(
        functools.partial(_combine_body, relu=relu),
        out_shape=jax.ShapeDtypeStruct((N, D), jnp.float32),
    )(accp, dp3, h, asv, adv, c, b2d)


def _edge_body(src_hbm, dst_hbm, h_hbm, asrc_hbm, adst_hbm, c_hbm,
               accp_hbm, dp_hbm,
               asrc_v, adst_v, c_v, srcb, dstb, ee_v, rows, zrow, zd,
               acc_sh, den_sh, sem):
    cid = lax.axis_index("c")
    sid = lax.axis_index("s")
    wid = cid * NS + sid

    zv = jnp.zeros((16,), jnp.float32)

    @pl.loop(0, CHUNK)
    def _zero_rows(r):
        for j in range(D // 16):
            zrow[r, pl.ds(j * 16, 16)] = zv

    @pl.loop(0, STRIPE, step=16)
    def _zero_zd(i):
        zd[pl.ds(i, 16)] = zv

    base = sid * STRIPE
    for k in range(STRIPE // CHUNK):
        pltpu.sync_copy(zrow, acc_sh.at[pl.ds(base + k * CHUNK, CHUNK)])
    pltpu.sync_copy(zd, den_sh.at[pl.ds(base, STRIPE)])

    pltpu.sync_copy(asrc_hbm, asrc_v)
    pltpu.sync_copy(adst_hbm, adst_v)
    pltpu.sync_copy(c_hbm, c_v)

    plsc.subcore_barrier()

    cvec = c_v[...]
    ebase = wid * EPW

    @pl.loop(0, NCHUNK)
    def _edges(k):
        off = ebase + k * CHUNK
        pltpu.sync_copy(src_hbm.at[pl.ds(off, CHUNK)], srcb)
        pltpu.sync_copy(dst_hbm.at[pl.ds(off, CHUNK)], dstb)
        gat = pltpu.async_copy(h_hbm.at[srcb], rows, sem)

        for v in range(CHUNK // 16):
            sl = pl.ds(v * 16, 16)
            a_s = plsc.load_gather(asrc_v, [srcb[sl]])
            a_d = plsc.load_gather(adst_v, [dstb[sl]])
            e = a_s + a_d
            e = jnp.where(e >= 0.0, e, 0.2 * e)
            ee_v[sl] = jnp.exp(e - cvec)

        pltpu.sync_copy(ee_v, den_sh.at[dstb], add=True)
        gat.wait()

        @pl.loop(0, CHUNK)
        def _scale(r):
            eeb = plsc.load_gather(ee_v, [jnp.full((16,), 0, jnp.int32) + r])
            for j in range(D // 16):
                sl = pl.ds(j * 16, 16)
                rows[r, sl] = rows[r, sl] * eeb

        pltpu.sync_copy(rows, acc_sh.at[dstb], add=True)

    plsc.subcore_barrier()

    pltpu.sync_copy(acc_sh.at[pl.ds(base, STRIPE)],
                    accp_hbm.at[cid, pl.ds(base, STRIPE)])
    pltpu.sync_copy(den_sh.at[pl.ds(base, STRIPE)],
                    dp_hbm.at[cid, pl.ds(base, STRIPE)])


_SC_PARAMS = pltpu.CompilerParams()
if "needs_layout_passes" in pltpu.CompilerParams.__dataclass_fields__:
    _SC_PARAMS = dataclasses.replace(_SC_PARAMS, needs_layout_passes=False)


def _edge_pass(src, dst, h, asrc, adst, cvec16):
    mesh = plsc.VectorSubcoreMesh(core_axis_name="c", subcore_axis_name="s")
    k = pl.kernel(
        _edge_body,
        out_type=(
            jax.ShapeDtypeStruct((NC, NPAD, D), jnp.float32),
            jax.ShapeDtypeStruct((NC, NPAD), jnp.float32),
        ),
        mesh=mesh,
        scratch_types=[
            pltpu.VMEM((N,), jnp.float32),
            pltpu.VMEM((N,), jnp.float32),
            pltpu.VMEM((16,), jnp.float32),
            pltpu.VMEM((CHUNK,), jnp.int32),
            pltpu.VMEM((CHUNK,), jnp.int32),
            pltpu.VMEM((CHUNK,), jnp.float32),
            pltpu.VMEM((CHUNK, D), jnp.float32),
            pltpu.VMEM((CHUNK, D), jnp.float32),
            pltpu.VMEM((STRIPE,), jnp.float32),
            pltpu.VMEM_SHARED((NPAD, D), jnp.float32),
            pltpu.VMEM_SHARED((NPAD,), jnp.float32),
            pltpu.SemaphoreType.DMA,
        ],
        compiler_params=_SC_PARAMS,
    )
    return k(src, dst, h, asrc, adst, cvec16)


def kernel(x, edge_index, W1, a_src1, a_dst1, b1, W2, a_src2, a_dst2, b2):
    src = edge_index[0]
    dst = edge_index[1]
    asv1 = a_src1.reshape(D, 1)
    adv1 = a_dst1.reshape(D, 1)
    asv2 = a_src2.reshape(D, 1)
    adv2 = a_dst2.reshape(D, 1)

    h1, as1, ad1, c1 = _proj(x, W1, asv1, adv1)
    c1v = jnp.broadcast_to(c1.reshape(1), (16,))
    accp1, dp1 = _edge_pass(src, dst, h1, as1.reshape(N), ad1.reshape(N), c1v)
    z1 = _combine(accp1, dp1.reshape(NC, NPAD, 1), h1, asv1, adv1, c1,
                  b1.reshape(1, D), relu=True)

    h2, as2, ad2, c2 = _proj(z1, W2, asv2, adv2)
    c2v = jnp.broadcast_to(c2.reshape(1), (16,))
    accp2, dp2 = _edge_pass(src, dst, h2, as2.reshape(N), ad2.reshape(N), c2v)
    out = _combine(accp2, dp2.reshape(NC, NPAD, 1), h2, asv2, adv2, c2,
                   b2.reshape(1, D), relu=False)
    return out


# speedup vs baseline: 33.4429x; 1.4076x over previous
import dataclasses
import functools

import jax
import jax.numpy as jnp
from jax import lax
from jax.experimental import pallas as pl
from jax.experimental.pallas import tpu as pltpu
from jax.experimental.pallas import tpu_sc as plsc

N = 10000
D = 128
E = 320000
NC = 2
NS = 16
NW = NC * NS
EPW = E // NW
CHUNK = 96
PADW = 80
EPWP = EPW + PADW
NCHUNK = EPWP // CHUNK
RING = 4
NPAD = 10240
STRIPE = NPAD // NS


def _proj_body(x_ref, w_ref, asv_ref, adv_ref, h_ref, as_ref, ad_ref, c_ref):
    h = jnp.dot(x_ref[...], w_ref[...], preferred_element_type=jnp.float32)
    h_ref[...] = h
    a_s = jnp.dot(h, asv_ref[...], preferred_element_type=jnp.float32)
    a_d = jnp.dot(h, adv_ref[...], preferred_element_type=jnp.float32)
    as_ref[...] = a_s
    ad_ref[...] = a_d
    c = jnp.maximum(jnp.max(a_s) + jnp.max(a_d), 0.0)
    c_ref[...] = jnp.reshape(c, (1, 1))


def _proj(x, w, asv, adv):
    return pl.pallas_call(
        _proj_body,
        out_shape=(
            jax.ShapeDtypeStruct((N, D), jnp.float32),
            jax.ShapeDtypeStruct((N, 1), jnp.float32),
            jax.ShapeDtypeStruct((N, 1), jnp.float32),
            jax.ShapeDtypeStruct((1, 1), jnp.float32),
        ),
    )(x, w, asv, adv)


def _combine_body(accp_ref, dp_ref, h_ref, asv_ref, adv_ref, c_ref, b_ref,
                  z_ref, *, relu):
    h = h_ref[...]
    es = (jnp.dot(h, asv_ref[...], preferred_element_type=jnp.float32)
          + jnp.dot(h, adv_ref[...], preferred_element_type=jnp.float32))
    es = jnp.where(es >= 0.0, es, 0.2 * es)
    es = jnp.exp(es - c_ref[...])
    acc = accp_ref[0, :N, :] + accp_ref[1, :N, :] + es * h
    den = dp_ref[0, :N, :] + dp_ref[1, :N, :] + es + 1e-16
    z = acc / den + b_ref[...]
    if relu:
        z = jnp.maximum(z, 0.0)
    z_ref[...] = z


def _combine(accp, dp3, h, asv, adv, c, b2d, relu):
    return pl.pallas_call(
        functools.partial(_combine_body, relu=relu),
        out_shape=jax.ShapeDtypeStruct((N, D), jnp.float32),
    )(accp, dp3, h, asv, adv, c, b2d)


def _edge_body(src_hbm, dst_hbm, h_hbm, asrc_hbm, adst_hbm, c_hbm,
               accp_hbm, dp_hbm, *s):
    asrc_v, adst_v, c_v = s[0], s[1], s[2]
    src_i = s[3:7]
    dst_i = s[7:11]
    ee_v = s[11:15]
    rows = s[15:17]
    zd = s[17]
    acc_sh = s[18]
    den_sh = s[19]
    sem_idx = s[20]
    sem_gat = s[21:23]
    sem_se = s[23:27]
    sem_sr = s[27:29]

    cid = lax.axis_index("c")
    sid = lax.axis_index("s")
    wid = cid * NS + sid

    zv = jnp.zeros((16,), jnp.float32)

    @pl.loop(0, CHUNK)
    def _zero_rows(r):
        for j in range(D // 16):
            rows[0][r, pl.ds(j * 16, 16)] = zv

    @pl.loop(0, STRIPE, step=16)
    def _zero_zd(i):
        zd[pl.ds(i, 16)] = zv

    base = sid * STRIPE
    for t in range(STRIPE // CHUNK):
        pltpu.sync_copy(rows[0], acc_sh.at[pl.ds(base + t * CHUNK, CHUNK)])
    rem = STRIPE % CHUNK
    if rem:
        pltpu.sync_copy(
            rows[0].at[pl.ds(0, rem)],
            acc_sh.at[pl.ds(base + (STRIPE // CHUNK) * CHUNK, rem)])
    pltpu.sync_copy(zd, den_sh.at[pl.ds(base, STRIPE)])

    pltpu.sync_copy(asrc_hbm, asrc_v)
    pltpu.sync_copy(adst_hbm, adst_v)
    pltpu.sync_copy(c_hbm, c_v)

    cvec = c_v[...]
    ebase = wid * EPWP

    pltpu.sync_copy(src_hbm.at[pl.ds(ebase, CHUNK)], src_i[0])
    pltpu.sync_copy(dst_hbm.at[pl.ds(ebase, CHUNK)], dst_i[0])
    pltpu.async_copy(h_hbm.at[src_i[0]], rows[0], sem_gat[0])

    plsc.subcore_barrier()

    def step(k, b2, b4, drain_ee, drain_row, prefetch):
        nb4 = (b4 + 1) % RING
        pb4 = (b4 + 3) % RING
        pb2 = 1 - b2
        if drain_ee:
            pltpu.make_async_copy(
                ee_v[nb4], den_sh.at[dst_i[nb4]], sem_se[nb4]).wait()
        if prefetch:
            off = ebase + (k + 1) * CHUNK
            pltpu.async_copy(src_hbm.at[pl.ds(off, CHUNK)], src_i[nb4], sem_idx)
            pltpu.async_copy(dst_hbm.at[pl.ds(off, CHUNK)], dst_i[nb4], sem_idx)

        for v in range(CHUNK // 16):
            sl = pl.ds(v * 16, 16)
            a_s = plsc.load_gather(asrc_v, [src_i[b4][sl]])
            a_d = plsc.load_gather(adst_v, [dst_i[b4][sl]])
            e = a_s + a_d
            e = jnp.where(e >= 0.0, e, 0.2 * e)
            ee_v[b4][sl] = jnp.exp(e - cvec)

        if drain_row:
            pltpu.make_async_copy(
                rows[pb2], acc_sh.at[dst_i[pb4]], sem_sr[pb2]).wait()
        if prefetch:
            off = ebase + (k + 1) * CHUNK
            pltpu.make_async_copy(
                src_hbm.at[pl.ds(off, CHUNK)], src_i[nb4], sem_idx).wait()
            pltpu.make_async_copy(
                dst_hbm.at[pl.ds(off, CHUNK)], dst_i[nb4], sem_idx).wait()
            pltpu.async_copy(h_hbm.at[src_i[nb4]], rows[pb2], sem_gat[pb2])

        pltpu.async_copy(ee_v[b4], den_sh.at[dst_i[b4]], sem_se[b4], add=True)
        pltpu.make_async_copy(h_hbm.at[src_i[b4]], rows[b2], sem_gat[b2]).wait()

        @pl.loop(0, CHUNK)
        def _scale(r):
            eeb = plsc.load_gather(ee_v[b4],
                                   [jnp.full((16,), 0, jnp.int32) + r])
            for j in range(D // 16):
                sl = pl.ds(j * 16, 16)
                rows[b2][r, sl] = rows[b2][r, sl] * eeb

        pltpu.async_copy(rows[b2], acc_sh.at[dst_i[b4]], sem_sr[b2], add=True)

    step(0, 0, 0, False, False, True)
    step(1, 1, 1, False, True, True)
    step(2, 0, 2, False, True, True)

    MAIN_END = 3 + ((NCHUNK - 4 - 3) // RING) * RING

    @pl.loop(3, MAIN_END, step=RING)
    def _main(g):
        for j in range(RING):
            step(g + j, (3 + j) % 2, (3 + j) % RING, True, True, True)

    for k in range(MAIN_END, NCHUNK):
        step(k, k % 2, k % RING, True, True, k < NCHUNK - 1)

    pltpu.make_async_copy(
        rows[(NCHUNK - 1) % 2],
        acc_sh.at[dst_i[(NCHUNK - 1) % RING]],
        sem_sr[(NCHUNK - 1) % 2]).wait()
    for kk in (NCHUNK - 3, NCHUNK - 2, NCHUNK - 1):
        pltpu.make_async_copy(
            ee_v[kk % RING], den_sh.at[dst_i[kk % RING]],
            sem_se[kk % RING]).wait()

    plsc.subcore_barrier()

    pltpu.sync_copy(acc_sh.at[pl.ds(base, STRIPE)],
                    accp_hbm.at[cid, pl.ds(base, STRIPE)])
    pltpu.sync_copy(den_sh.at[pl.ds(base, STRIPE)],
                    dp_hbm.at[cid, pl.ds(base, STRIPE)])


_SC_PARAMS = pltpu.CompilerParams()
if "needs_layout_passes" in pltpu.CompilerParams.__dataclass_fields__:
    _SC_PARAMS = dataclasses.replace(_SC_PARAMS, needs_layout_passes=False)


def _edge_pass(src, dst, h, asrc, adst_pad, cvec16):
    mesh = plsc.VectorSubcoreMesh(core_axis_name="c", subcore_axis_name="s")
    scratch = [
        pltpu.VMEM((N,), jnp.float32),
        pltpu.VMEM((NPAD,), jnp.float32),
        pltpu.VMEM((16,), jnp.float32),
    ]
    scratch += [pltpu.VMEM((CHUNK,), jnp.int32) for _ in range(RING)]
    scratch += [pltpu.VMEM((CHUNK,), jnp.int32) for _ in range(RING)]
    scratch += [pltpu.VMEM((CHUNK,), jnp.float32) for _ in range(RING)]
    scratch += [pltpu.VMEM((CHUNK, D), jnp.float32) for _ in range(2)]
    scratch += [
        pltpu.VMEM((STRIPE,), jnp.float32),
        pltpu.VMEM_SHARED((NPAD, D), jnp.float32),
        pltpu.VMEM_SHARED((NPAD,), jnp.float32),
        pltpu.SemaphoreType.DMA,
        pltpu.SemaphoreType.DMA,
        pltpu.SemaphoreType.DMA,
    ]
    scratch += [pltpu.SemaphoreType.DMA for _ in range(RING)]
    scratch += [pltpu.SemaphoreType.DMA for _ in range(2)]
    k = pl.kernel(
        _edge_body,
        out_type=(
            jax.ShapeDtypeStruct((NC, NPAD, D), jnp.float32),
            jax.ShapeDtypeStruct((NC, NPAD), jnp.float32),
        ),
        mesh=mesh,
        scratch_types=scratch,
        compiler_params=_SC_PARAMS,
    )
    return k(src, dst, h, asrc, adst_pad, cvec16)


def kernel(x, edge_index, W1, a_src1, a_dst1, b1, W2, a_src2, a_dst2, b2):
    src = edge_index[0]
    dst = edge_index[1]
    srcp = jnp.pad(src.reshape(NW, EPW), ((0, 0), (0, PADW))).reshape(-1)
    dstp = jnp.pad(dst.reshape(NW, EPW), ((0, 0), (0, PADW)),
                   constant_values=N).reshape(-1)
    neg_tail = jnp.full((NPAD - N,), -1e30, jnp.float32)

    asv1 = a_src1.reshape(D, 1)
    adv1 = a_dst1.reshape(D, 1)
    asv2 = a_src2.reshape(D, 1)
    adv2 = a_dst2.reshape(D, 1)

    h1, as1, ad1, c1 = _proj(x, W1, asv1, adv1)
    c1v = jnp.broadcast_to(c1.reshape(1), (16,))
    ad1p = jnp.concatenate([ad1.reshape(N), neg_tail])
    accp1, dp1 = _edge_pass(srcp, dstp, h1, as1.reshape(N), ad1p, c1v)
    z1 = _combine(accp1, dp1.reshape(NC, NPAD, 1), h1, asv1, adv1, c1,
                  b1.reshape(1, D), relu=True)

    h2, as2, ad2, c2 = _proj(z1, W2, asv2, adv2)
    c2v = jnp.broadcast_to(c2.reshape(1), (16,))
    ad2p = jnp.concatenate([ad2.reshape(N), neg_tail])
    accp2, dp2 = _edge_pass(srcp, dstp, h2, as2.reshape(N), ad2p, c2v)
    out = _combine(accp2, dp2.reshape(NC, NPAD, 1), h2, asv2, adv2, c2,
                   b2.reshape(1, D), relu=False)
    return out


# speedup vs baseline: 36.6154x; 1.0949x over previous
import dataclasses
import functools

import jax
import jax.numpy as jnp
from jax import lax
from jax.experimental import pallas as pl
from jax.experimental.pallas import tpu as pltpu
from jax.experimental.pallas import tpu_sc as plsc

N = 10000
D = 128
E = 320000
NC = 2
NS = 16
NW = NC * NS
EPW = E // NW
CHUNK = 96
PADW = 80
EPWP = EPW + PADW
NCHUNK = EPWP // CHUNK
HALF = CHUNK // 2
RING = 4
NPAD = 10240
STRIPE = NPAD // NS


def _proj_body(x_ref, w_ref, asv_ref, adv_ref, h_ref, as_ref, ad_ref, c_ref):
    h = jnp.dot(x_ref[...], w_ref[...], preferred_element_type=jnp.float32)
    h_ref[...] = h
    a_s = jnp.dot(h, asv_ref[...], preferred_element_type=jnp.float32)
    a_d = jnp.dot(h, adv_ref[...], preferred_element_type=jnp.float32)
    as_ref[...] = a_s
    ad_ref[...] = a_d
    c = jnp.maximum(jnp.max(a_s) + jnp.max(a_d), 0.0)
    c_ref[...] = jnp.reshape(c, (1, 1))


def _proj(x, w, asv, adv):
    return pl.pallas_call(
        _proj_body,
        out_shape=(
            jax.ShapeDtypeStruct((N, D), jnp.float32),
            jax.ShapeDtypeStruct((N, 1), jnp.float32),
            jax.ShapeDtypeStruct((N, 1), jnp.float32),
            jax.ShapeDtypeStruct((1, 1), jnp.float32),
        ),
    )(x, w, asv, adv)


def _combine_body(accp_ref, dp_ref, h_ref, asv_ref, adv_ref, c_ref, b_ref,
                  z_ref, *, relu):
    h = h_ref[...]
    es = (jnp.dot(h, asv_ref[...], preferred_element_type=jnp.float32)
          + jnp.dot(h, adv_ref[...], preferred_element_type=jnp.float32))
    es = jnp.where(es >= 0.0, es, 0.2 * es)
    es = jnp.exp(es - c_ref[...])
    acc = accp_ref[0, :N, :] + accp_ref[1, :N, :] + es * h
    den = dp_ref[0, :N, :] + dp_ref[1, :N, :] + es + 1e-16
    z = acc / den + b_ref[...]
    if relu:
        z = jnp.maximum(z, 0.0)
    z_ref[...] = z


def _combine(accp, dp3, h, asv, adv, c, b2d, relu):
    return pl.pallas_call(
        functools.partial(_combine_body, relu=relu),
        out_shape=jax.ShapeDtypeStruct((N, D), jnp.float32),
    )(accp, dp3, h, asv, adv, c, b2d)


def _edge_body(src_hbm, dst_hbm, h_hbm, asrc_hbm, adst_hbm, c_hbm,
               accp_hbm, dp_hbm, *s):
    asrc_v, adst_v, c_v = s[0], s[1], s[2]
    src_i = s[3:7]
    dst_i = s[7:11]
    ee_v = s[11:15]
    rows = s[15:17]
    zd = s[17]
    acc_sh = s[18]
    den_sh = s[19]
    sem_idx = s[20]
    sem_gat = s[21:23]
    sem_se = s[23:27]
    sem_sr = s[27:29]

    cid = lax.axis_index("c")
    sid = lax.axis_index("s")
    wid = cid * NS + sid

    zv = jnp.zeros((16,), jnp.float32)

    @pl.loop(0, CHUNK)
    def _zero_rows(r):
        for j in range(D // 16):
            rows[0][r, pl.ds(j * 16, 16)] = zv

    @pl.loop(0, STRIPE, step=16)
    def _zero_zd(i):
        zd[pl.ds(i, 16)] = zv

    base = sid * STRIPE
    for t in range(STRIPE // CHUNK):
        pltpu.sync_copy(rows[0], acc_sh.at[pl.ds(base + t * CHUNK, CHUNK)])
    rem = STRIPE % CHUNK
    if rem:
        pltpu.sync_copy(
            rows[0].at[pl.ds(0, rem)],
            acc_sh.at[pl.ds(base + (STRIPE // CHUNK) * CHUNK, rem)])
    pltpu.sync_copy(zd, den_sh.at[pl.ds(base, STRIPE)])

    pltpu.sync_copy(asrc_hbm, asrc_v)
    pltpu.sync_copy(adst_hbm, adst_v)
    pltpu.sync_copy(c_hbm, c_v)

    cvec = c_v[...]
    ebase = wid * EPWP

    pltpu.sync_copy(src_hbm.at[pl.ds(ebase, HALF)], src_i[0].at[0])
    pltpu.sync_copy(src_hbm.at[pl.ds(ebase + HALF, HALF)], src_i[0].at[1])
    pltpu.sync_copy(dst_hbm.at[pl.ds(ebase, HALF)], dst_i[0].at[0])
    pltpu.sync_copy(dst_hbm.at[pl.ds(ebase + HALF, HALF)], dst_i[0].at[1])
    pltpu.async_copy(h_hbm.at[src_i[0].at[0]], rows[0].at[pl.ds(0, HALF)],
                     sem_gat[0])
    pltpu.async_copy(h_hbm.at[src_i[0].at[1]], rows[0].at[pl.ds(HALF, HALF)],
                     sem_gat[0])

    plsc.subcore_barrier()

    def step(k, b2, b4, drain_ee, drain_row, prefetch):
        nb4 = (b4 + 1) % RING
        pb4 = (b4 + 3) % RING
        pb2 = 1 - b2
        if drain_ee:
            for hf in range(2):
                pltpu.make_async_copy(
                    ee_v[nb4].at[pl.ds(hf * HALF, HALF)],
                    den_sh.at[dst_i[nb4].at[hf]], sem_se[nb4]).wait()
        if prefetch:
            off = ebase + (k + 1) * CHUNK
            for hf in range(2):
                pltpu.async_copy(src_hbm.at[pl.ds(off + hf * HALF, HALF)],
                                 src_i[nb4].at[hf], sem_idx)
                pltpu.async_copy(dst_hbm.at[pl.ds(off + hf * HALF, HALF)],
                                 dst_i[nb4].at[hf], sem_idx)

        for v in range(CHUNK // 16):
            sl = pl.ds(v * 16, 16)
            hi, hs = divmod(v * 16, HALF)
            a_s = plsc.load_gather(asrc_v, [src_i[b4][hi, pl.ds(hs, 16)]])
            a_d = plsc.load_gather(adst_v, [dst_i[b4][hi, pl.ds(hs, 16)]])
            e = a_s + a_d
            e = jnp.where(e >= 0.0, e, 0.2 * e)
            ee_v[b4][sl] = jnp.exp(e - cvec)

        if drain_row:
            for hf in range(2):
                pltpu.make_async_copy(
                    rows[pb2].at[pl.ds(hf * HALF, HALF)],
                    acc_sh.at[dst_i[pb4].at[hf]], sem_sr[pb2]).wait()
        if prefetch:
            off = ebase + (k + 1) * CHUNK
            for hf in range(2):
                pltpu.make_async_copy(
                    src_hbm.at[pl.ds(off + hf * HALF, HALF)],
                    src_i[nb4].at[hf], sem_idx).wait()
                pltpu.make_async_copy(
                    dst_hbm.at[pl.ds(off + hf * HALF, HALF)],
                    dst_i[nb4].at[hf], sem_idx).wait()
            for hf in range(2):
                pltpu.async_copy(h_hbm.at[src_i[nb4].at[hf]],
                                 rows[pb2].at[pl.ds(hf * HALF, HALF)],
                                 sem_gat[pb2])

        for hf in range(2):
            pltpu.async_copy(ee_v[b4].at[pl.ds(hf * HALF, HALF)],
                             den_sh.at[dst_i[b4].at[hf]], sem_se[b4], add=True)

        for hf in range(2):
            pltpu.make_async_copy(
                h_hbm.at[src_i[b4].at[hf]],
                rows[b2].at[pl.ds(hf * HALF, HALF)], sem_gat[b2]).wait()

            @pl.loop(0, HALF, step=2)
            def _scale(r, _hf=hf):
                rr = _hf * HALF + r
                e0 = plsc.load_gather(ee_v[b4],
                                      [jnp.full((16,), 0, jnp.int32) + rr])
                e1 = plsc.load_gather(ee_v[b4],
                                      [jnp.full((16,), 1, jnp.int32) + rr])
                for j in range(D // 16):
                    sl = pl.ds(j * 16, 16)
                    rows[b2][rr, sl] = rows[b2][rr, sl] * e0
                    rows[b2][rr + 1, sl] = rows[b2][rr + 1, sl] * e1

            pltpu.async_copy(rows[b2].at[pl.ds(hf * HALF, HALF)],
                             acc_sh.at[dst_i[b4].at[hf]], sem_sr[b2], add=True)

    step(0, 0, 0, False, False, True)
    step(1, 1, 1, False, True, True)
    step(2, 0, 2, False, True, True)

    MAIN_END = 3 + ((NCHUNK - 4 - 3) // RING) * RING

    @pl.loop(3, MAIN_END, step=RING)
    def _main(g):
        for j in range(RING):
            step(g + j, (3 + j) % 2, (3 + j) % RING, True, True, True)

    for k in range(MAIN_END, NCHUNK):
        step(k, k % 2, k % RING, True, True, k < NCHUNK - 1)

    for hf in range(2):
        pltpu.make_async_copy(
            rows[(NCHUNK - 1) % 2].at[pl.ds(hf * HALF, HALF)],
            acc_sh.at[dst_i[(NCHUNK - 1) % RING].at[hf]],
            sem_sr[(NCHUNK - 1) % 2]).wait()
    for kk in (NCHUNK - 3, NCHUNK - 2, NCHUNK - 1):
        for hf in range(2):
            pltpu.make_async_copy(
                ee_v[kk % RING].at[pl.ds(hf * HALF, HALF)],
                den_sh.at[dst_i[kk % RING].at[hf]],
                sem_se[kk % RING]).wait()

    plsc.subcore_barrier()

    pltpu.sync_copy(acc_sh.at[pl.ds(base, STRIPE)],
                    accp_hbm.at[cid, pl.ds(base, STRIPE)])
    pltpu.sync_copy(den_sh.at[pl.ds(base, STRIPE)],
                    dp_hbm.at[cid, pl.ds(base, STRIPE)])


_SC_PARAMS = pltpu.CompilerParams()
if "needs_layout_passes" in pltpu.CompilerParams.__dataclass_fields__:
    _SC_PARAMS = dataclasses.replace(_SC_PARAMS, needs_layout_passes=False)


def _edge_pass(src, dst, h, asrc, adst_pad, cvec16):
    mesh = plsc.VectorSubcoreMesh(core_axis_name="c", subcore_axis_name="s")
    scratch = [
        pltpu.VMEM((N,), jnp.float32),
        pltpu.VMEM((NPAD,), jnp.float32),
        pltpu.VMEM((16,), jnp.float32),
    ]
    scratch += [pltpu.VMEM((2, HALF), jnp.int32) for _ in range(RING)]
    scratch += [pltpu.VMEM((2, HALF), jnp.int32) for _ in range(RING)]
    scratch += [pltpu.VMEM((CHUNK,), jnp.float32) for _ in range(RING)]
    scratch += [pltpu.VMEM((CHUNK, D), jnp.float32) for _ in range(2)]
    scratch += [
        pltpu.VMEM((STRIPE,), jnp.float32),
        pltpu.VMEM_SHARED((NPAD, D), jnp.float32),
        pltpu.VMEM_SHARED((NPAD,), jnp.float32),
        pltpu.SemaphoreType.DMA,
        pltpu.SemaphoreType.DMA,
        pltpu.SemaphoreType.DMA,
    ]
    scratch += [pltpu.SemaphoreType.DMA for _ in range(RING)]
    scratch += [pltpu.SemaphoreType.DMA for _ in range(2)]
    k = pl.kernel(
        _edge_body,
        out_type=(
            jax.ShapeDtypeStruct((NC, NPAD, D), jnp.float32),
            jax.ShapeDtypeStruct((NC, NPAD), jnp.float32),
        ),
        mesh=mesh,
        scratch_types=scratch,
        compiler_params=_SC_PARAMS,
    )
    return k(src, dst, h, asrc, adst_pad, cvec16)


def kernel(x, edge_index, W1, a_src1, a_dst1, b1, W2, a_src2, a_dst2, b2):
    src = edge_index[0]
    dst = edge_index[1]
    srcp = jnp.pad(src.reshape(NW, EPW), ((0, 0), (0, PADW))).reshape(-1)
    dstp = jnp.pad(dst.reshape(NW, EPW), ((0, 0), (0, PADW)),
                   constant_values=N).reshape(-1)
    neg_tail = jnp.full((NPAD - N,), -1e30, jnp.float32)

    asv1 = a_src1.reshape(D, 1)
    adv1 = a_dst1.reshape(D, 1)
    asv2 = a_src2.reshape(D, 1)
    adv2 = a_dst2.reshape(D, 1)

    h1, as1, ad1, c1 = _proj(x, W1, asv1, adv1)
    c1v = jnp.broadcast_to(c1.reshape(1), (16,))
    ad1p = jnp.concatenate([ad1.reshape(N), neg_tail])
    accp1, dp1 = _edge_pass(srcp, dstp, h1, as1.reshape(N), ad1p, c1v)
    z1 = _combine(accp1, dp1.reshape(NC, NPAD, 1), h1, asv1, adv1, c1,
                  b1.reshape(1, D), relu=True)

    h2, as2, ad2, c2 = _proj(z1, W2, asv2, adv2)
    c2v = jnp.broadcast_to(c2.reshape(1), (16,))
    ad2p = jnp.concatenate([ad2.reshape(N), neg_tail])
    accp2, dp2 = _edge_pass(srcp, dstp, h2, as2.reshape(N), ad2p, c2v)
    out = _combine(accp2, dp2.reshape(NC, NPAD, 1), h2, asv2, adv2, c2,
                   b2.reshape(1, D), relu=False)
    return out
